# baseline (speedup 1.0000x reference)
import functools

import jax
import jax.numpy as jnp
from jax import lax
from jax.experimental import pallas as pl
from jax.experimental.pallas import tpu as pltpu
from jax.experimental.pallas import tpu_sc as plsc

_NC = 2
_NS = 16
_NW = _NC * _NS
_CHUNK = 128
_D = 64
_LS = 32768
_NROW = 1000000
_NQ = 253952


def _round16(x):
    return lax.bitcast_convert_type(x, jnp.int32) + jnp.int32(0x8000)


def _detile_body(i_ref, o_ref):
    xT = i_ref[...].T
    p = _LS // 4
    a = jnp.concatenate([xT[:p], xT[p:2 * p]], axis=1)
    b = jnp.concatenate([xT[2 * p:3 * p], xT[3 * p:]], axis=1)
    ia = _round16(a)
    ib = _round16(b)
    o_ref[...] = (ib & jnp.int32(0xFFFF0000 - (1 << 32))) | (
        (ia >> 16) & jnp.int32(0xFFFF))


def _detile(table):
    t2 = table.T
    nstrip = (_NROW + _LS - 1) // _LS
    return pl.pallas_call(
        _detile_body,
        grid=(nstrip,),
        in_specs=[pl.BlockSpec((_D, _LS), lambda s: (0, s))],
        out_specs=pl.BlockSpec((_LS // 4, 2 * _D), lambda s: (s, 0)),
        out_shape=jax.ShapeDtypeStruct((_NQ, 2 * _D), jnp.int32),
    )(t2)


def _gather_body(n_chunks, b_per_w,
                 uq_hbm, iq_hbm, tdu_hbm, tdi_hbm,
                 uout_hbm, iout_hbm,
                 idx_v, rows_v, sem):
    wid = lax.axis_index("s") * _NC + lax.axis_index("c")
    row0 = wid * n_chunks
    base = wid * b_per_w
    for idx_hbm, td_hbm, out_hbm in (
        (uq_hbm, tdu_hbm, uout_hbm),
        (iq_hbm, tdi_hbm, iout_hbm),
    ):
        pltpu.sync_copy(idx_hbm.at[pl.ds(row0, n_chunks)], idx_v)
        copies = []
        for j in range(n_chunks):
            copies.append(pltpu.async_copy(
                td_hbm.at[idx_v.at[j]],
                rows_v.at[pl.ds(j * _CHUNK, _CHUNK)], sem))
        for c in copies:
            c.wait()
        pltpu.sync_copy(rows_v, out_hbm.at[pl.ds(base, b_per_w)])


def _sc_gather(uq2, iq2, td_u, td_i):
    b = uq2.shape[0] * uq2.shape[1]
    b_per_w = b // _NW
    n_chunks = b_per_w // _CHUNK
    mesh = plsc.VectorSubcoreMesh(core_axis_name="c", subcore_axis_name="s")
    k = pl.kernel(
        functools.partial(_gather_body, n_chunks, b_per_w),
        mesh=mesh,
        out_type=[
            jax.ShapeDtypeStruct((b, 2 * _D), jnp.int32),
            jax.ShapeDtypeStruct((b, 2 * _D), jnp.int32),
        ],
        scratch_types=[
            pltpu.VMEM((n_chunks, _CHUNK), jnp.int32),
            pltpu.VMEM((b_per_w, 2 * _D), jnp.int32),
            pltpu.SemaphoreType.DMA,
        ],
    )
    return k(uq2, iq2, td_u, td_i)


def _mlp_body(gu_ref, gi_ref, uh_ref, ih_ref, w1a_ref, w1b_ref, b1_ref,
              w2_ref, b2_ref, w3_ref, b3_ref, o_ref):
    def unpack_select(g32, qs):
        lo = lax.bitcast_convert_type(g32 << 16, jnp.float32)
        hi = lax.bitcast_convert_type(
            g32 & jnp.int32(0xFFFF0000 - (1 << 32)), jnp.float32)
        return jnp.where(
            qs < 2,
            jnp.where(qs == 0, lo[:, :_D], lo[:, _D:]),
            jnp.where(qs == 2, hi[:, :_D], hi[:, _D:]),
        )

    u = unpack_select(gu_ref[...], uh_ref[...])
    i = unpack_select(gi_ref[...], ih_ref[...])
    h = jnp.dot(u, w1a_ref[...], preferred_element_type=jnp.float32)
    h = h + jnp.dot(i, w1b_ref[...], preferred_element_type=jnp.float32)
    h = jnp.maximum(h + b1_ref[...], 0.0)
    h = jnp.dot(h, w2_ref[...], preferred_element_type=jnp.float32) + b2_ref[...]
    h = jnp.maximum(h, 0.0)
    o_ref[...] = jnp.dot(h, w3_ref[...], preferred_element_type=jnp.float32) + b3_ref[...]


def _tc_mlp(gu, gi, uh, ih, W1a, W1b, b1, W2, b2, W3, b3):
    b = gu.shape[0]
    blk = 2048
    grid = (b // blk,)
    row_spec = pl.BlockSpec((blk, 2 * _D), lambda g: (g, 0))
    col_spec = pl.BlockSpec((blk, 1), lambda g: (g, 0))
    full = lambda shape: pl.BlockSpec(shape, lambda g: (0, 0))
    return pl.pallas_call(
        _mlp_body,
        grid=grid,
        in_specs=[
            row_spec, row_spec, col_spec, col_spec,
            full(W1a.shape), full(W1b.shape), full(b1.shape),
            full(W2.shape), full(b2.shape),
            full(W3.shape), full(b3.shape),
        ],
        out_specs=pl.BlockSpec((blk, 1), lambda g: (g, 0)),
        out_shape=jax.ShapeDtypeStruct((b, 1), jnp.float32),
    )(gu, gi, uh, ih, W1a, W1b, b1, W2, b2, W3, b3)


def kernel(user_id, item_id, user_table, item_table, W1, b1, W2, b2, W3, b3):
    b = user_id.shape[0]
    uid = user_id.astype(jnp.int32)
    iid = item_id.astype(jnp.int32)
    sh = _LS.bit_length() - 1
    qm = _LS // 4 - 1
    uq = ((uid >> sh) << (sh - 2)) + (uid & qm)
    iq = ((iid >> sh) << (sh - 2)) + (iid & qm)
    uh = ((uid >> (sh - 2)) & 3).reshape(b, 1)
    ih = ((iid >> (sh - 2)) & 3).reshape(b, 1)
    uq2 = uq.reshape(b // _CHUNK, _CHUNK)
    iq2 = iq.reshape(b // _CHUNK, _CHUNK)
    td_u = _detile(user_table)
    td_i = _detile(item_table)
    gu, gi = _sc_gather(uq2, iq2, td_u, td_i)
    return _tc_mlp(
        gu, gi, uh, ih,
        W1[:_D], W1[_D:], b1.reshape(1, -1),
        W2, b2.reshape(1, -1),
        W3, b3.reshape(1, -1),
    )


# speedup vs baseline: 1.0036x; 1.0036x over previous
import functools

import jax
import jax.numpy as jnp
from jax import lax
from jax.experimental import pallas as pl
from jax.experimental.pallas import tpu as pltpu
from jax.experimental.pallas import tpu_sc as plsc

_NC = 2
_NS = 16
_NW = _NC * _NS
_CHUNK = 128
_D = 64
_LS = 32768
_NROW = 1000000
_NQ = 253952


def _detile_body(i_ref, o_ref):
    xT = i_ref[...].T
    p = _LS // 4
    a = jnp.concatenate([xT[:p], xT[p:2 * p]], axis=1)
    b = jnp.concatenate([xT[2 * p:3 * p], xT[3 * p:]], axis=1)
    ua = lax.bitcast_convert_type(a, jnp.uint32) >> 16
    ub = lax.bitcast_convert_type(b, jnp.uint32) & jnp.uint32(0xFFFF0000)
    o_ref[...] = lax.bitcast_convert_type(ua | ub, jnp.int32)


def _detile(table):
    t2 = table.T
    nstrip = (_NROW + _LS - 1) // _LS
    return pl.pallas_call(
        _detile_body,
        grid=(nstrip,),
        in_specs=[pl.BlockSpec((_D, _LS), lambda s: (0, s))],
        out_specs=pl.BlockSpec((_LS // 4, 2 * _D), lambda s: (s, 0)),
        out_shape=jax.ShapeDtypeStruct((_NQ, 2 * _D), jnp.int32),
    )(t2)


def _gather_body(n_chunks, b_per_w,
                 uq_hbm, iq_hbm, tdu_hbm, tdi_hbm,
                 uout_hbm, iout_hbm,
                 idx_v, rows_v, sem):
    wid = lax.axis_index("s") * _NC + lax.axis_index("c")
    row0 = wid * n_chunks
    base = wid * b_per_w
    for idx_hbm, td_hbm, out_hbm in (
        (uq_hbm, tdu_hbm, uout_hbm),
        (iq_hbm, tdi_hbm, iout_hbm),
    ):
        pltpu.sync_copy(idx_hbm.at[pl.ds(row0, n_chunks)], idx_v)
        copies = []
        for j in range(n_chunks):
            copies.append(pltpu.async_copy(
                td_hbm.at[idx_v.at[j]],
                rows_v.at[pl.ds(j * _CHUNK, _CHUNK)], sem))
        for c in copies:
            c.wait()
        pltpu.sync_copy(rows_v, out_hbm.at[pl.ds(base, b_per_w)])


def _sc_gather(uq2, iq2, td_u, td_i):
    b = uq2.shape[0] * uq2.shape[1]
    b_per_w = b // _NW
    n_chunks = b_per_w // _CHUNK
    mesh = plsc.VectorSubcoreMesh(core_axis_name="c", subcore_axis_name="s")
    k = pl.kernel(
        functools.partial(_gather_body, n_chunks, b_per_w),
        mesh=mesh,
        out_type=[
            jax.ShapeDtypeStruct((b, 2 * _D), jnp.int32),
            jax.ShapeDtypeStruct((b, 2 * _D), jnp.int32),
        ],
        scratch_types=[
            pltpu.VMEM((n_chunks, _CHUNK), jnp.int32),
            pltpu.VMEM((b_per_w, 2 * _D), jnp.int32),
            pltpu.SemaphoreType.DMA,
        ],
    )
    return k(uq2, iq2, td_u, td_i)


def _mlp_body(gu_ref, gi_ref, uh_ref, ih_ref, w1a_ref, w1b_ref, b1_ref,
              w2_ref, b2_ref, w3_ref, b3_ref, o_ref):
    def unpack_select(g32, qs):
        lo = lax.bitcast_convert_type(g32 << 16, jnp.float32)
        hi = lax.bitcast_convert_type(
            g32 & jnp.int32(0xFFFF0000 - (1 << 32)), jnp.float32)
        return jnp.where(
            qs < 2,
            jnp.where(qs == 0, lo[:, :_D], lo[:, _D:]),
            jnp.where(qs == 2, hi[:, :_D], hi[:, _D:]),
        )

    u = unpack_select(gu_ref[...], uh_ref[...])
    i = unpack_select(gi_ref[...], ih_ref[...])
    h = jnp.dot(u, w1a_ref[...], preferred_element_type=jnp.float32)
    h = h + jnp.dot(i, w1b_ref[...], preferred_element_type=jnp.float32)
    h = jnp.maximum(h + b1_ref[...], 0.0)
    h = jnp.dot(h, w2_ref[...], preferred_element_type=jnp.float32) + b2_ref[...]
    h = jnp.maximum(h, 0.0)
    o_ref[...] = jnp.dot(h, w3_ref[...], preferred_element_type=jnp.float32) + b3_ref[...]


def _tc_mlp(gu, gi, uh, ih, W1a, W1b, b1, W2, b2, W3, b3):
    b = gu.shape[0]
    blk = 2048
    grid = (b // blk,)
    row_spec = pl.BlockSpec((blk, 2 * _D), lambda g: (g, 0))
    col_spec = pl.BlockSpec((blk, 1), lambda g: (g, 0))
    full = lambda shape: pl.BlockSpec(shape, lambda g: (0, 0))
    return pl.pallas_call(
        _mlp_body,
        grid=grid,
        in_specs=[
            row_spec, row_spec, col_spec, col_spec,
            full(W1a.shape), full(W1b.shape), full(b1.shape),
            full(W2.shape), full(b2.shape),
            full(W3.shape), full(b3.shape),
        ],
        out_specs=pl.BlockSpec((blk, 1), lambda g: (g, 0)),
        out_shape=jax.ShapeDtypeStruct((b, 1), jnp.float32),
    )(gu, gi, uh, ih, W1a, W1b, b1, W2, b2, W3, b3)


def kernel(user_id, item_id, user_table, item_table, W1, b1, W2, b2, W3, b3):
    b = user_id.shape[0]
    uid = user_id.astype(jnp.int32)
    iid = item_id.astype(jnp.int32)
    sh = _LS.bit_length() - 1
    qm = _LS // 4 - 1
    uq = ((uid >> sh) << (sh - 2)) + (uid & qm)
    iq = ((iid >> sh) << (sh - 2)) + (iid & qm)
    uh = ((uid >> (sh - 2)) & 3).reshape(b, 1)
    ih = ((iid >> (sh - 2)) & 3).reshape(b, 1)
    uq2 = uq.reshape(b // _CHUNK, _CHUNK)
    iq2 = iq.reshape(b // _CHUNK, _CHUNK)
    td_u = _detile(user_table)
    td_i = _detile(item_table)
    gu, gi = _sc_gather(uq2, iq2, td_u, td_i)
    return _tc_mlp(
        gu, gi, uh, ih,
        W1[:_D], W1[_D:], b1.reshape(1, -1),
        W2, b2.reshape(1, -1),
        W3, b3.reshape(1, -1),
    )


# speedup vs baseline: 1.0145x; 1.0109x over previous
import functools

import jax
import jax.numpy as jnp
from jax import lax
from jax.experimental import pallas as pl
from jax.experimental.pallas import tpu as pltpu
from jax.experimental.pallas import tpu_sc as plsc

_NC = 2
_NS = 16
_NW = _NC * _NS
_CHUNK = 128
_D = 64
_LS = 32768
_NROW = 1000000
_NQ = 507904


def _detile_body(i_ref, o_ref):
    xT = i_ref[...].T
    h = _LS // 2
    o_ref[...] = jnp.concatenate([xT[:h], xT[h:]], axis=1)


def _detile(table):
    t2 = table.T
    nstrip = (_NROW + _LS - 1) // _LS
    return pl.pallas_call(
        _detile_body,
        grid=(nstrip,),
        in_specs=[pl.BlockSpec((_D, _LS), lambda s: (0, s))],
        out_specs=pl.BlockSpec((_LS // 2, 2 * _D), lambda s: (s, 0)),
        out_shape=jax.ShapeDtypeStruct((_NQ, 2 * _D), jnp.float32),
    )(t2)


def _gather_body(n_chunks, b_per_w,
                 uq_hbm, iq_hbm, tdu_hbm, tdi_hbm,
                 uout_hbm, iout_hbm,
                 idx_v, rows_v, sem):
    wid = lax.axis_index("s") * _NC + lax.axis_index("c")
    row0 = wid * n_chunks
    base = wid * b_per_w
    for idx_hbm, td_hbm, out_hbm in (
        (uq_hbm, tdu_hbm, uout_hbm),
        (iq_hbm, tdi_hbm, iout_hbm),
    ):
        pltpu.sync_copy(idx_hbm.at[pl.ds(row0, n_chunks)], idx_v)
        copies = []
        for j in range(n_chunks):
            copies.append(pltpu.async_copy(
                td_hbm.at[idx_v.at[j]],
                rows_v.at[pl.ds(j * _CHUNK, _CHUNK)], sem))
        for c in copies:
            c.wait()
        pltpu.sync_copy(rows_v, out_hbm.at[pl.ds(base, b_per_w)])


def _sc_gather(uq2, iq2, td_u, td_i):
    b = uq2.shape[0] * uq2.shape[1]
    b_per_w = b // _NW
    n_chunks = b_per_w // _CHUNK
    mesh = plsc.VectorSubcoreMesh(core_axis_name="c", subcore_axis_name="s")
    k = pl.kernel(
        functools.partial(_gather_body, n_chunks, b_per_w),
        mesh=mesh,
        out_type=[
            jax.ShapeDtypeStruct((b, 2 * _D), jnp.float32),
            jax.ShapeDtypeStruct((b, 2 * _D), jnp.float32),
        ],
        scratch_types=[
            pltpu.VMEM((n_chunks, _CHUNK), jnp.int32),
            pltpu.VMEM((b_per_w, 2 * _D), jnp.float32),
            pltpu.SemaphoreType.DMA,
        ],
    )
    return k(uq2, iq2, td_u, td_i)


def _mlp_body(gu_ref, gi_ref, uh_ref, ih_ref, w1a_ref, w1b_ref, b1_ref,
              w2_ref, b2_ref, w3_ref, b3_ref, o_ref):
    gu = gu_ref[...]
    gi = gi_ref[...]
    u = jnp.where(uh_ref[...] == 0, gu[:, :_D], gu[:, _D:])
    i = jnp.where(ih_ref[...] == 0, gi[:, :_D], gi[:, _D:])
    h = jnp.dot(u, w1a_ref[...], preferred_element_type=jnp.float32)
    h = h + jnp.dot(i, w1b_ref[...], preferred_element_type=jnp.float32)
    h = jnp.maximum(h + b1_ref[...], 0.0)
    h = jnp.dot(h, w2_ref[...], preferred_element_type=jnp.float32) + b2_ref[...]
    h = jnp.maximum(h, 0.0)
    o_ref[...] = jnp.dot(h, w3_ref[...], preferred_element_type=jnp.float32) + b3_ref[...]


def _tc_mlp(gu, gi, uh, ih, W1a, W1b, b1, W2, b2, W3, b3):
    b = gu.shape[0]
    blk = 2048
    grid = (b // blk,)
    row_spec = pl.BlockSpec((blk, 2 * _D), lambda g: (g, 0))
    col_spec = pl.BlockSpec((blk, 1), lambda g: (g, 0))
    full = lambda shape: pl.BlockSpec(shape, lambda g: (0, 0))
    return pl.pallas_call(
        _mlp_body,
        grid=grid,
        in_specs=[
            row_spec, row_spec, col_spec, col_spec,
            full(W1a.shape), full(W1b.shape), full(b1.shape),
            full(W2.shape), full(b2.shape),
            full(W3.shape), full(b3.shape),
        ],
        out_specs=pl.BlockSpec((blk, 1), lambda g: (g, 0)),
        out_shape=jax.ShapeDtypeStruct((b, 1), jnp.float32),
    )(gu, gi, uh, ih, W1a, W1b, b1, W2, b2, W3, b3)


def kernel(user_id, item_id, user_table, item_table, W1, b1, W2, b2, W3, b3):
    b = user_id.shape[0]
    uid = user_id.astype(jnp.int32)
    iid = item_id.astype(jnp.int32)
    sh = _LS.bit_length() - 1
    hm = _LS // 2 - 1
    uq = ((uid >> sh) << (sh - 1)) + (uid & hm)
    iq = ((iid >> sh) << (sh - 1)) + (iid & hm)
    uh = ((uid >> (sh - 1)) & 1).reshape(b, 1)
    ih = ((iid >> (sh - 1)) & 1).reshape(b, 1)
    uq2 = uq.reshape(b // _CHUNK, _CHUNK)
    iq2 = iq.reshape(b // _CHUNK, _CHUNK)
    td_u = _detile(user_table)
    td_i = _detile(item_table)
    gu, gi = _sc_gather(uq2, iq2, td_u, td_i)
    return _tc_mlp(
        gu, gi, uh, ih,
        W1[:_D], W1[_D:], b1.reshape(1, -1),
        W2, b2.reshape(1, -1),
        W3, b3.reshape(1, -1),
    )


# speedup vs baseline: 1.0169x; 1.0024x over previous
import functools

import jax
import jax.numpy as jnp
from jax import lax
from jax.experimental import pallas as pl
from jax.experimental.pallas import tpu as pltpu
from jax.experimental.pallas import tpu_sc as plsc

_NC = 2
_NS = 16
_NW = _NC * _NS
_CHUNK = 128
_D = 64
_LS = 32768
_NROW = 1000000
_NQ = 507904


def _detile_body(i_ref, o_ref):
    xT = i_ref[...].T
    h = _LS // 2
    o_ref[...] = jnp.concatenate([xT[:h], xT[h:]], axis=1)


def _detile(table):
    t2 = table.T
    nstrip = (_NROW + _LS - 1) // _LS
    return pl.pallas_call(
        _detile_body,
        grid=(nstrip,),
        in_specs=[pl.BlockSpec((_D, _LS), lambda s: (0, s))],
        out_specs=pl.BlockSpec((_LS // 2, 2 * _D), lambda s: (s, 0)),
        out_shape=jax.ShapeDtypeStruct((_NQ, 2 * _D), jnp.float32),
    )(t2)


def _gather_body(n_chunks, b_per_w,
                 idx_hbm, td_hbm, out_hbm,
                 idx_v, rows_v, sem):
    wid = lax.axis_index("s") * _NC + lax.axis_index("c")
    row0 = wid * n_chunks
    base = wid * b_per_w
    pltpu.sync_copy(idx_hbm.at[pl.ds(row0, n_chunks)], idx_v)
    copies = []
    for j in range(n_chunks):
        copies.append(pltpu.async_copy(
            td_hbm.at[idx_v.at[j]],
            rows_v.at[pl.ds(j * _CHUNK, _CHUNK)], sem))
    for c in copies:
        c.wait()
    pltpu.sync_copy(rows_v, out_hbm.at[pl.ds(base, b_per_w)])


def _sc_gather(q2, td):
    b = q2.shape[0] * q2.shape[1]
    b_per_w = b // _NW
    n_chunks = b_per_w // _CHUNK
    mesh = plsc.VectorSubcoreMesh(core_axis_name="c", subcore_axis_name="s")
    k = pl.kernel(
        functools.partial(_gather_body, n_chunks, b_per_w),
        mesh=mesh,
        out_type=jax.ShapeDtypeStruct((b, 2 * _D), jnp.float32),
        scratch_types=[
            pltpu.VMEM((n_chunks, _CHUNK), jnp.int32),
            pltpu.VMEM((b_per_w, 2 * _D), jnp.float32),
            pltpu.SemaphoreType.DMA,
        ],
    )
    return k(q2, td)


def _mlp_body(gu_ref, gi_ref, uh_ref, ih_ref, w1a_ref, w1b_ref, b1_ref,
              w2_ref, b2_ref, w3_ref, b3_ref, o_ref):
    gu = gu_ref[...]
    gi = gi_ref[...]
    u = jnp.where(uh_ref[...] == 0, gu[:, :_D], gu[:, _D:])
    i = jnp.where(ih_ref[...] == 0, gi[:, :_D], gi[:, _D:])
    h = jnp.dot(u, w1a_ref[...], preferred_element_type=jnp.float32)
    h = h + jnp.dot(i, w1b_ref[...], preferred_element_type=jnp.float32)
    h = jnp.maximum(h + b1_ref[...], 0.0)
    h = jnp.dot(h, w2_ref[...], preferred_element_type=jnp.float32) + b2_ref[...]
    h = jnp.maximum(h, 0.0)
    o_ref[...] = jnp.dot(h, w3_ref[...], preferred_element_type=jnp.float32) + b3_ref[...]


def _tc_mlp(gu, gi, uh, ih, W1a, W1b, b1, W2, b2, W3, b3):
    b = gu.shape[0]
    blk = 2048
    grid = (b // blk,)
    row_spec = pl.BlockSpec((blk, 2 * _D), lambda g: (g, 0))
    col_spec = pl.BlockSpec((blk, 1), lambda g: (g, 0))
    full = lambda shape: pl.BlockSpec(shape, lambda g: (0, 0))
    return pl.pallas_call(
        _mlp_body,
        grid=grid,
        in_specs=[
            row_spec, row_spec, col_spec, col_spec,
            full(W1a.shape), full(W1b.shape), full(b1.shape),
            full(W2.shape), full(b2.shape),
            full(W3.shape), full(b3.shape),
        ],
        out_specs=pl.BlockSpec((blk, 1), lambda g: (g, 0)),
        out_shape=jax.ShapeDtypeStruct((b, 1), jnp.float32),
    )(gu, gi, uh, ih, W1a, W1b, b1, W2, b2, W3, b3)


def kernel(user_id, item_id, user_table, item_table, W1, b1, W2, b2, W3, b3):
    b = user_id.shape[0]
    uid = user_id.astype(jnp.int32)
    iid = item_id.astype(jnp.int32)
    sh = _LS.bit_length() - 1
    hm = _LS // 2 - 1
    uq = ((uid >> sh) << (sh - 1)) + (uid & hm)
    iq = ((iid >> sh) << (sh - 1)) + (iid & hm)
    uh = ((uid >> (sh - 1)) & 1).reshape(b, 1)
    ih = ((iid >> (sh - 1)) & 1).reshape(b, 1)
    uq2 = uq.reshape(b // _CHUNK, _CHUNK)
    iq2 = iq.reshape(b // _CHUNK, _CHUNK)
    td_u = _detile(user_table)
    gu = _sc_gather(uq2, td_u)
    td_i = _detile(item_table)
    gi = _sc_gather(iq2, td_i)
    return _tc_mlp(
        gu, gi, uh, ih,
        W1[:_D], W1[_D:], b1.reshape(1, -1),
        W2, b2.reshape(1, -1),
        W3, b3.reshape(1, -1),
    )
